# baseline (speedup 1.0000x reference)
import functools

import jax
import jax.numpy as jnp
import numpy as np
from jax.experimental import pallas as pl
from jax.experimental.pallas import tpu as pltpu

_TINY = np.float32(np.finfo(np.float32).tiny)
_SPAN = np.float32(np.float32(1.0) - _TINY)
_KS0 = np.uint32(0)
_KS1 = np.uint32(42)
_KS2 = np.uint32(_KS0 ^ _KS1 ^ np.uint32(0x1BD11BDA))
_ROT_A = (13, 15, 26, 6)
_ROT_B = (17, 29, 16, 24)
_BIG = np.int32(2**30)


def _rotl(x, r):
    return jax.lax.shift_left(x, np.uint32(r)) | jax.lax.shift_right_logical(
        x, np.uint32(32 - r))


def _four_rounds(x0, x1, rots):
    for r in rots:
        x0 = x0 + x1
        x1 = _rotl(x1, r) ^ x0
    return x0, x1


def _threefry_bits(i):
    x0 = jnp.zeros_like(i) + _KS0
    x1 = i + _KS1
    x0, x1 = _four_rounds(x0, x1, _ROT_A)
    x0 = x0 + _KS1
    x1 = x1 + np.uint32(_KS2 + np.uint32(1))
    x0, x1 = _four_rounds(x0, x1, _ROT_B)
    x0 = x0 + _KS2
    x1 = x1 + np.uint32(_KS0 + np.uint32(2))
    x0, x1 = _four_rounds(x0, x1, _ROT_A)
    x0 = x0 + _KS0
    x1 = x1 + np.uint32(_KS1 + np.uint32(3))
    x0, x1 = _four_rounds(x0, x1, _ROT_B)
    x0 = x0 + _KS1
    x1 = x1 + np.uint32(_KS2 + np.uint32(4))
    x0, x1 = _four_rounds(x0, x1, _ROT_A)
    x0 = x0 + _KS2
    x1 = x1 + np.uint32(_KS0 + np.uint32(5))
    return x0 ^ x1


def _uniform_from_bits(bits):
    fb = jax.lax.shift_right_logical(bits, np.uint32(9)) | np.uint32(0x3F800000)
    f = jax.lax.bitcast_convert_type(fb, jnp.float32) - np.float32(1.0)
    return jnp.maximum(_TINY, f * _SPAN + _TINY)


def _merge_top2(av, ai, ax, au, bv, bi, bx, bu):
    a1v, a2v = av[:, 0:1], av[:, 1:2]
    b1v, b2v = bv[:, 0:1], bv[:, 1:2]
    a_first = (a1v > b1v) | ((a1v == b1v) & (ai[:, 0:1] < bi[:, 0:1]))

    def sel(c, t, f):
        return jnp.where(c, t, f)

    n1v = sel(a_first, a1v, b1v)
    n1i = sel(a_first, ai[:, 0:1], bi[:, 0:1])
    n1x = sel(a_first, ax[:, 0:1], bx[:, 0:1])
    n1u = sel(a_first, au[:, 0:1], bu[:, 0:1])
    c2v = sel(a_first, b1v, a1v)
    c2i = sel(a_first, bi[:, 0:1], ai[:, 0:1])
    c2x = sel(a_first, bx[:, 0:1], ax[:, 0:1])
    c2u = sel(a_first, bu[:, 0:1], au[:, 0:1])
    d2v = sel(a_first, a2v, b2v)
    d2i = sel(a_first, ai[:, 1:2], bi[:, 1:2])
    d2x = sel(a_first, ax[:, 1:2], bx[:, 1:2])
    d2u = sel(a_first, au[:, 1:2], bu[:, 1:2])
    c_first = (c2v > d2v) | ((c2v == d2v) & (c2i < d2i))
    n2v = sel(c_first, c2v, d2v)
    n2i = sel(c_first, c2i, d2i)
    n2x = sel(c_first, c2x, d2x)
    n2u = sel(c_first, c2u, d2u)
    return (jnp.concatenate([n1v, n2v], axis=1),
            jnp.concatenate([n1i, n2i], axis=1),
            jnp.concatenate([n1x, n2x], axis=1),
            jnp.concatenate([n1u, n2u], axis=1))


def _block_top2(v, x, u, col):
    v1 = jnp.max(v, axis=1, keepdims=True)
    i1 = jnp.min(jnp.where(v == v1, col, _BIG), axis=1, keepdims=True)
    hit1 = col == i1
    x1 = jnp.max(jnp.where(hit1, x, -jnp.inf), axis=1, keepdims=True)
    u1 = jnp.max(jnp.where(hit1, u, -jnp.inf), axis=1, keepdims=True)
    vm = jnp.where(hit1, -jnp.inf, v)
    v2 = jnp.max(vm, axis=1, keepdims=True)
    i2 = jnp.min(jnp.where(vm == v2, col, _BIG), axis=1, keepdims=True)
    hit2 = col == i2
    x2 = jnp.max(jnp.where(hit2, x, -jnp.inf), axis=1, keepdims=True)
    u2 = jnp.max(jnp.where(hit2, u, -jnp.inf), axis=1, keepdims=True)
    return (jnp.concatenate([v1, v2], axis=1),
            jnp.concatenate([i1, i2], axis=1),
            jnp.concatenate([x1, x2], axis=1),
            jnp.concatenate([u1, u2], axis=1))


def _tc_shard_kernel(logits_ref, statsf_ref, statsi_ref, *, width, vocab,
                     nblocks, rows):
    j = pl.program_id(0)

    @pl.when(j == 0)
    def _init():
        c8 = jax.lax.broadcasted_iota(jnp.int32, (rows, 8), 1)
        init = jnp.where(c8 == 1, np.float32(0.0),
                         jnp.where((c8 == 4) | (c8 == 7), np.float32(0.5),
                                   np.float32(-np.inf)))
        statsf_ref[...] = init
        statsi_ref[...] = jnp.full((rows, 8), _BIG, dtype=jnp.int32)

    x_raw = logits_ref[...]
    col = j * width + jax.lax.broadcasted_iota(jnp.int32, x_raw.shape, 1)
    valid = col < vocab
    x = jnp.where(valid, x_raw, -jnp.inf)
    row = jax.lax.broadcasted_iota(jnp.int32, x_raw.shape, 0)
    flat = (row * vocab + col).astype(jnp.uint32)
    u = _uniform_from_bits(_threefry_bits(flat))
    g = -jnp.log(-jnp.log(u))
    v = x + g

    sf = statsf_ref[...]
    si = statsi_ref[...]
    m_old = sf[:, 0:1]
    z_old = sf[:, 1:2]
    m_blk = jnp.max(x, axis=1, keepdims=True)
    m_new = jnp.maximum(m_old, m_blk)
    z_new = z_old * jnp.exp(m_old - m_new) + jnp.sum(
        jnp.exp(x - m_new), axis=1, keepdims=True)

    bv, bi, bx, bu = _block_top2(v, x, u, col)
    rv = jnp.concatenate([sf[:, 2:3], sf[:, 5:6]], axis=1)
    ri = si[:, 0:2]
    rx = jnp.concatenate([sf[:, 3:4], sf[:, 6:7]], axis=1)
    ru = jnp.concatenate([sf[:, 4:5], sf[:, 7:8]], axis=1)
    nv, ni, nx, nu = _merge_top2(rv, ri, rx, ru, bv, bi, bx, bu)

    statsf_ref[...] = jnp.concatenate([
        m_new, z_new, nv[:, 0:1], nx[:, 0:1], nu[:, 0:1], nv[:, 1:2],
        nx[:, 1:2], nu[:, 1:2]], axis=1)
    statsi_ref[...] = jnp.concatenate(
        [ni, jnp.full((rows, 6), _BIG, dtype=jnp.int32)], axis=1)


def _run_tc_shard(logits, width=16384):
    rows, vocab = logits.shape
    nblocks = (vocab + width - 1) // width
    kern = functools.partial(_tc_shard_kernel, width=width, vocab=vocab,
                             nblocks=nblocks, rows=rows)
    statsf, statsi = pl.pallas_call(
        kern,
        grid=(nblocks,),
        in_specs=[pl.BlockSpec((rows, width), lambda j: (0, j))],
        out_specs=[pl.BlockSpec((rows, 8), lambda j: (0, 0)),
                   pl.BlockSpec((rows, 8), lambda j: (0, 0))],
        out_shape=[jax.ShapeDtypeStruct((rows, 8), jnp.float32),
                   jax.ShapeDtypeStruct((rows, 8), jnp.int32)],
        compiler_params=pltpu.CompilerParams(
            dimension_semantics=("arbitrary",)),
    )(logits)
    return statsf, statsi


def kernel(logits):
    rows, vocab = logits.shape
    statsf, statsi = _run_tc_shard(logits)
    m = statsf[:, 0]
    z = statsf[:, 1]
    cx = jnp.stack([statsf[:, 3], statsf[:, 6]], axis=1)
    cu = jnp.stack([statsf[:, 4], statsf[:, 7]], axis=1)
    ci = statsi[:, 0:2]
    cu = jnp.clip(cu, _TINY, np.float32(1.0 - 2.0**-24))
    gumb = -jnp.log(-jnp.log(cu))
    p = jnp.exp(cx - m[:, None]) / z[:, None]
    zscore = gumb + jnp.log(p)
    zbest = jnp.max(zscore, axis=1, keepdims=True)
    best = jnp.min(jnp.where(zscore == zbest, ci, _BIG), axis=1)
    return best.astype(jnp.int32)
